# Initial kernel scaffold; baseline (speedup 1.0000x reference)
import jax
import jax.numpy as jnp
from jax.experimental import pallas as pl


def kernel(h, x, edge_index, edge_attr, params):
    raise NotImplementedError("write your pallas kernel here")


# speedup vs baseline: 2.8500x; 2.8500x over previous
import functools

import jax
import jax.numpy as jnp
from jax import lax
from jax.experimental import pallas as pl
from jax.experimental.pallas import tpu as pltpu
from jax.experimental.pallas import tpu_sc as plsc

N = 50000
E = 800000
EP = 802816
NR = EP // 128
NC, NS = 2, 16
NW = NC * NS
RPW = NR // NW
G = 4
STEPS = RPW // G
GS = 2
STEPS_S = RPW // GS
GD = 36
MD = 37
NPS = N // NS
ZR = 125
BE = 2048
BN = 2000

def _mesh():
    return plsc.VectorSubcoreMesh(core_axis_name="c", subcore_axis_name="s")


_SC_PARAMS = functools.partial(
    pltpu.CompilerParams, use_tc_tiling_on_sc=False)


def _silu(v):
    return v * jax.nn.sigmoid(v)


def _sc_gather(tab, row2, col2):
    def body(tab_r, row_r, col_r, gr_o, gc_o, rbuf, cbuf, grb, gcb, sem):
        wid = lax.axis_index("s") * NC + lax.axis_index("c")
        base = wid * RPW

        def step(g, carry):
            cid = base + g * G
            pltpu.sync_copy(row_r.at[pl.ds(cid, G)], rbuf)
            pltpu.sync_copy(col_r.at[pl.ds(cid, G)], cbuf)
            cps = []
            for j in range(G):
                cps.append(pltpu.async_copy(
                    tab_r.at[rbuf.at[j]], grb.at[pl.ds(j * 128, 128)], sem))
                cps.append(pltpu.async_copy(
                    tab_r.at[cbuf.at[j]], gcb.at[pl.ds(j * 128, 128)], sem))
            for cp in cps:
                cp.wait()
            e0 = cid * 128
            pltpu.sync_copy(grb, gr_o.at[pl.ds(e0, G * 128)])
            pltpu.sync_copy(gcb, gc_o.at[pl.ds(e0, G * 128)])
            return carry

        lax.fori_loop(0, STEPS, step, 0)

    return pl.kernel(
        body,
        out_type=(jax.ShapeDtypeStruct((EP, GD), jnp.float32),
                  jax.ShapeDtypeStruct((EP, GD), jnp.float32)),
        mesh=_mesh(),
        compiler_params=_SC_PARAMS(),
        scratch_types=[
            pltpu.VMEM((G, 128), jnp.int32),
            pltpu.VMEM((G, 128), jnp.int32),
            pltpu.VMEM((G * 128, GD), jnp.float32),
            pltpu.VMEM((G * 128, GD), jnp.float32),
            pltpu.SemaphoreType.DMA,
        ],
    )(tab, row2, col2)


def _sc_scatter(mt, row2, zrows):
    def body(mt_r, row_r, z_r, out_r, acc, rbuf, mtb, sem):
        c = lax.axis_index("c")
        s = lax.axis_index("s")
        wid = s * NC + c
        r0 = s * NPS

        pltpu.sync_copy(z_r, mtb.at[pl.ds(0, ZR)])

        def zstep(k, carry):
            pltpu.sync_copy(mtb.at[pl.ds(0, ZR)],
                            acc.at[pl.ds(r0 + k * ZR, ZR)])
            return carry

        lax.fori_loop(0, NPS // ZR, zstep, 0)
        plsc.subcore_barrier()

        base = wid * RPW

        def step(g, carry):
            cid = base + g * GS
            pltpu.sync_copy(row_r.at[pl.ds(cid, GS)], rbuf)
            pltpu.sync_copy(mt_r.at[pl.ds(cid * 128, GS * 128)], mtb)
            for j in range(GS):
                pltpu.sync_copy(mtb.at[pl.ds(j * 128, 128)],
                                acc.at[rbuf.at[j]], add=True)
            return carry

        lax.fori_loop(0, STEPS_S, step, 0)
        plsc.subcore_barrier()

        def ostep(k, carry):
            pltpu.sync_copy(acc.at[pl.ds(r0 + k * ZR, ZR)],
                            mtb.at[pl.ds(0, ZR)])
            pltpu.sync_copy(mtb.at[pl.ds(0, ZR)],
                            out_r.at[c, pl.ds(r0 + k * ZR, ZR)])
            return carry

        lax.fori_loop(0, NPS // ZR, ostep, 0)

    return pl.kernel(
        body,
        out_type=jax.ShapeDtypeStruct((NC, N, MD), jnp.float32),
        mesh=_mesh(),
        compiler_params=_SC_PARAMS(),
        scratch_types=[
            pltpu.VMEM_SHARED((N, MD), jnp.float32),
            pltpu.VMEM((GS, 128), jnp.int32),
            pltpu.VMEM((GS * 128, MD), jnp.float32),
            pltpu.SemaphoreType.DMA,
        ],
    )(mt, row2, zrows)


def _tc_embed(h, w, b):
    def body(h_r, w_r, b_r, o_r):
        o_r[:] = jnp.dot(h_r[:], w_r[:],
                         preferred_element_type=jnp.float32) + b_r[:]

    return pl.pallas_call(
        body,
        grid=(N // BN,),
        in_specs=[
            pl.BlockSpec((BN, 125), lambda i: (i, 0)),
            pl.BlockSpec((125, 32), lambda i: (0, 0)),
            pl.BlockSpec((1, 32), lambda i: (0, 0)),
        ],
        out_specs=pl.BlockSpec((BN, 32), lambda i: (i, 0)),
        out_shape=jax.ShapeDtypeStruct((N, 32), jnp.float32),
    )(h, w, b)


def _tc_edge(gr, gc, ea, ws, wt, wr, we, b1, e2, b2, c1, c1b, c2t):
    def body(gr_r, gc_r, ea_r, ws_r, wt_r, wr_r, we_r, b1_r, e2_r, b2_r,
             c1_r, c1b_r, c2t_r, mt_o):
        pid = pl.program_id(0)
        hr = gr_r[:, 0:32]
        xr = gr_r[:, 32:36]
        hc = gc_r[:, 0:32]
        xc = gc_r[:, 32:36]
        d = xr - xc
        radial = jnp.sum(d * d, axis=1, keepdims=True)
        pre = (jnp.dot(hr, ws_r[:], preferred_element_type=jnp.float32)
               + jnp.dot(hc, wt_r[:], preferred_element_type=jnp.float32)
               + jnp.dot(ea_r[:], we_r[:], preferred_element_type=jnp.float32)
               + radial * wr_r[:] + b1_r[:])
        m1 = _silu(pre)
        m = _silu(jnp.dot(m1, e2_r[:],
                          preferred_element_type=jnp.float32) + b2_r[:])
        t = _silu(jnp.dot(m, c1_r[:],
                          preferred_element_type=jnp.float32) + c1b_r[:])
        sc = jnp.sum(t * c2t_r[:], axis=1, keepdims=True)
        ids = pid * BE + lax.broadcasted_iota(jnp.int32, (BE, 1), 0)
        validf = (ids < E).astype(jnp.float32)
        mt = jnp.concatenate(
            [m, d * sc, jnp.ones((BE, 1), jnp.float32)], axis=1) * validf
        mt_o[:] = mt

    full = lambda shape: pl.BlockSpec(shape, lambda i: (0, 0))
    return pl.pallas_call(
        body,
        grid=(EP // BE,),
        in_specs=[
            pl.BlockSpec((BE, GD), lambda i: (i, 0)),
            pl.BlockSpec((BE, GD), lambda i: (i, 0)),
            pl.BlockSpec((BE, 16), lambda i: (i, 0)),
            full((32, 32)), full((32, 32)), full((1, 32)), full((16, 32)),
            full((1, 32)), full((32, 32)), full((1, 32)), full((32, 32)),
            full((1, 32)), full((1, 32)),
        ],
        out_specs=pl.BlockSpec((BE, MD), lambda i: (i, 0)),
        out_shape=jax.ShapeDtypeStruct((EP, MD), jnp.float32),
    )(gr, gc, ea, ws, wt, wr, we, b1, e2, b2, c1, c1b, c2t)


def _tc_node(h, x4, a0, a1, n1h, n1m, n1b, n2, n2b, wo, bo):
    def body(h_r, x_r, a0_r, a1_r, n1h_r, n1m_r, n1b_r, n2_r, n2b_r,
             wo_r, bo_r, h_o, x_o):
        magg = a0_r[:, 0:32] + a1_r[:, 0:32]
        tr = a0_r[:, 32:36] + a1_r[:, 32:36]
        cnt = a0_r[:, 36:37] + a1_r[:, 36:37]
        o = _silu(jnp.dot(h_r[:], n1h_r[:], preferred_element_type=jnp.float32)
                  + jnp.dot(magg, n1m_r[:], preferred_element_type=jnp.float32)
                  + n1b_r[:])
        hn = h_r[:] + jnp.dot(o, n2_r[:],
                              preferred_element_type=jnp.float32) + n2b_r[:]
        h_o[:] = jnp.dot(hn, wo_r[:],
                         preferred_element_type=jnp.float32) + bo_r[:]
        x_o[:] = x_r[:] + tr / jnp.clip(cnt, 1.0, None)

    full = lambda shape: pl.BlockSpec(shape, lambda i: (0, 0))
    return pl.pallas_call(
        body,
        grid=(N // BN,),
        in_specs=[
            pl.BlockSpec((BN, 32), lambda i: (i, 0)),
            pl.BlockSpec((BN, 4), lambda i: (i, 0)),
            pl.BlockSpec((BN, MD), lambda i: (i, 0)),
            pl.BlockSpec((BN, MD), lambda i: (i, 0)),
            full((32, 32)), full((32, 32)), full((1, 32)),
            full((32, 32)), full((1, 32)), full((32, 32)), full((1, 32)),
        ],
        out_specs=(pl.BlockSpec((BN, 32), lambda i: (i, 0)),
                   pl.BlockSpec((BN, 4), lambda i: (i, 0))),
        out_shape=(jax.ShapeDtypeStruct((N, 32), jnp.float32),
                   jax.ShapeDtypeStruct((N, 4), jnp.float32)),
    )(h, x4, a0, a1, n1h, n1m, n1b, n2, n2b, wo, bo)


def kernel(h, x, edge_index, edge_attr, params):
    p = params
    row = edge_index[0]
    col = edge_index[1]
    row2 = jnp.pad(row, (0, EP - E)).reshape(NR, 128)
    col2 = jnp.pad(col, (0, EP - E)).reshape(NR, 128)
    eap = jnp.pad(edge_attr, ((0, EP - E), (0, 0)))
    zrows = jnp.zeros((ZR, MD), jnp.float32)
    eye = jnp.eye(32, dtype=jnp.float32)
    zb = jnp.zeros((1, 32), jnp.float32)

    hcur = _tc_embed(h, p['emb_in_w'], p['emb_in_b'].reshape(1, 32))
    x4 = jnp.concatenate([x, jnp.zeros((N, 1), jnp.float32)], axis=1)

    for l in range(3):
        tab = jnp.concatenate([hcur, x4], axis=1)
        gr, gc = _sc_gather(tab, row2, col2)
        e1w = p[f'e1_w_{l}']
        mt = _tc_edge(
            gr, gc, eap,
            e1w[0:32], e1w[32:64], e1w[64:65], e1w[65:81],
            p[f'e1_b_{l}'].reshape(1, 32),
            p[f'e2_w_{l}'], p[f'e2_b_{l}'].reshape(1, 32),
            p[f'c1_w_{l}'], p[f'c1_b_{l}'].reshape(1, 32),
            p[f'c2_w_{l}'].reshape(1, 32))
        accp = _sc_scatter(mt, row2, zrows)
        n1w = p[f'n1_w_{l}']
        if l < 2:
            wo, bo = eye, zb
        else:
            wo, bo = p['emb_out_w'], p['emb_out_b'].reshape(1, 32)
        hcur, x4 = _tc_node(
            hcur, x4, accp[0], accp[1],
            n1w[0:32], n1w[32:64], p[f'n1_b_{l}'].reshape(1, 32),
            p[f'n2_w_{l}'], p[f'n2_b_{l}'].reshape(1, 32), wo, bo)

    return hcur, x4[:, :3]


# speedup vs baseline: 3.7592x; 1.3190x over previous
import functools

import jax
import jax.numpy as jnp
from jax import lax
from jax.experimental import pallas as pl
from jax.experimental.pallas import tpu as pltpu
from jax.experimental.pallas import tpu_sc as plsc

N = 50000
E = 800000
EP = 802816
NR = EP // 128
NC, NS = 2, 16
NW = NC * NS
RPW = NR // NW
G = 4
STEPS = RPW // G
GS = 2
STEPS_S = RPW // GS
GD = 36
MD = 37
NPS = N // NS
ZR = 125
BE = 4096
BN = 2000

def _mesh():
    return plsc.VectorSubcoreMesh(core_axis_name="c", subcore_axis_name="s")


_SC_PARAMS = functools.partial(
    pltpu.CompilerParams, use_tc_tiling_on_sc=False)


def _silu(v):
    return 0.5 * v * jnp.tanh(0.5 * v) + 0.5 * v


def _sc_gather(tab, row2, col2):
    def body(tab_r, row_r, col_r, gr_o, gc_o, rbuf, cbuf, grb, gcb, sem):
        wid = lax.axis_index("s") * NC + lax.axis_index("c")
        base = wid * RPW

        def step(g, carry):
            cid = base + g * G
            pltpu.sync_copy(row_r.at[pl.ds(cid, G)], rbuf)
            pltpu.sync_copy(col_r.at[pl.ds(cid, G)], cbuf)
            cps = []
            for j in range(G):
                cps.append(pltpu.async_copy(
                    tab_r.at[rbuf.at[j]], grb.at[pl.ds(j * 128, 128)], sem))
                cps.append(pltpu.async_copy(
                    tab_r.at[cbuf.at[j]], gcb.at[pl.ds(j * 128, 128)], sem))
            for cp in cps:
                cp.wait()
            e0 = cid * 128
            pltpu.sync_copy(grb, gr_o.at[pl.ds(e0, G * 128)])
            pltpu.sync_copy(gcb, gc_o.at[pl.ds(e0, G * 128)])
            return carry

        lax.fori_loop(0, STEPS, step, 0)

    return pl.kernel(
        body,
        out_type=(jax.ShapeDtypeStruct((EP, GD), jnp.float32),
                  jax.ShapeDtypeStruct((EP, GD), jnp.float32)),
        mesh=_mesh(),
        compiler_params=_SC_PARAMS(),
        scratch_types=[
            pltpu.VMEM((G, 128), jnp.int32),
            pltpu.VMEM((G, 128), jnp.int32),
            pltpu.VMEM((G * 128, GD), jnp.float32),
            pltpu.VMEM((G * 128, GD), jnp.float32),
            pltpu.SemaphoreType.DMA,
        ],
    )(tab, row2, col2)


def _sc_scatter(mt, row2, zrows):
    def body(mt_r, row_r, z_r, out_r, acc, rbuf, mtb, sem):
        c = lax.axis_index("c")
        s = lax.axis_index("s")
        wid = s * NC + c
        r0 = s * NPS

        pltpu.sync_copy(z_r, mtb.at[pl.ds(0, ZR)])

        def zstep(k, carry):
            pltpu.sync_copy(mtb.at[pl.ds(0, ZR)],
                            acc.at[pl.ds(r0 + k * ZR, ZR)])
            return carry

        lax.fori_loop(0, NPS // ZR, zstep, 0)
        plsc.subcore_barrier()

        base = wid * RPW

        def step(g, carry):
            cid = base + g * GS
            pltpu.sync_copy(row_r.at[pl.ds(cid, GS)], rbuf)
            pltpu.sync_copy(mt_r.at[pl.ds(cid * 128, GS * 128)], mtb)
            for j in range(GS):
                pltpu.sync_copy(mtb.at[pl.ds(j * 128, 128)],
                                acc.at[rbuf.at[j]], add=True)
            return carry

        lax.fori_loop(0, STEPS_S, step, 0)
        plsc.subcore_barrier()

        def ostep(k, carry):
            pltpu.sync_copy(acc.at[pl.ds(r0 + k * ZR, ZR)],
                            mtb.at[pl.ds(0, ZR)])
            pltpu.sync_copy(mtb.at[pl.ds(0, ZR)],
                            out_r.at[c, pl.ds(r0 + k * ZR, ZR)])
            return carry

        lax.fori_loop(0, NPS // ZR, ostep, 0)

    return pl.kernel(
        body,
        out_type=jax.ShapeDtypeStruct((NC, N, MD), jnp.float32),
        mesh=_mesh(),
        compiler_params=_SC_PARAMS(),
        scratch_types=[
            pltpu.VMEM_SHARED((N, MD), jnp.float32),
            pltpu.VMEM((GS, 128), jnp.int32),
            pltpu.VMEM((GS * 128, MD), jnp.float32),
            pltpu.SemaphoreType.DMA,
        ],
    )(mt, row2, zrows)


def _tc_embed(h, x, w, b):
    def body(h_r, x_r, w_r, b_r, o_r):
        h0 = jnp.dot(h_r[:], w_r[:],
                     preferred_element_type=jnp.float32) + b_r[:]
        o_r[:] = jnp.concatenate(
            [h0, x_r[:], jnp.zeros((BN, 1), jnp.float32)], axis=1)

    return pl.pallas_call(
        body,
        grid=(N // BN,),
        in_specs=[
            pl.BlockSpec((BN, 125), lambda i: (i, 0)),
            pl.BlockSpec((BN, 3), lambda i: (i, 0)),
            pl.BlockSpec((125, 32), lambda i: (0, 0)),
            pl.BlockSpec((1, 32), lambda i: (0, 0)),
        ],
        out_specs=pl.BlockSpec((BN, GD), lambda i: (i, 0)),
        out_shape=jax.ShapeDtypeStruct((N, GD), jnp.float32),
    )(h, x, w, b)


def _tc_edge(grT, gcT, eaT, at, bt, wrc, wet, b1c, e2t, b2c, c1t, c1bc, c2r):
    def body(gr_r, gc_r, ea_r, at_r, bt_r, wrc_r, wet_r, b1c_r, e2t_r,
             b2c_r, c1t_r, c1bc_r, c2r_r, mt_o):
        pid = pl.program_id(0)
        grv = gr_r[:]
        gcv = gc_r[:]
        d = grv[32:36, :] - gcv[32:36, :]
        radial = jnp.sum(d * d, axis=0, keepdims=True)
        pre = (jnp.dot(at_r[:], grv, preferred_element_type=jnp.float32)
               + jnp.dot(bt_r[:], gcv, preferred_element_type=jnp.float32)
               + jnp.dot(wet_r[:], ea_r[:],
                         preferred_element_type=jnp.float32)
               + wrc_r[:] * radial + b1c_r[:])
        m1 = _silu(pre)
        m = _silu(jnp.dot(e2t_r[:], m1,
                          preferred_element_type=jnp.float32) + b2c_r[:])
        t = _silu(jnp.dot(c1t_r[:], m,
                          preferred_element_type=jnp.float32) + c1bc_r[:])
        sc = jnp.dot(c2r_r[:], t, preferred_element_type=jnp.float32)
        ids = pid * BE + lax.broadcasted_iota(jnp.int32, (1, BE), 1)
        validf = (ids < E).astype(jnp.float32)
        mt = jnp.concatenate(
            [m, d * sc, jnp.ones((1, BE), jnp.float32)], axis=0) * validf
        mt_o[:] = mt

    full = lambda shape: pl.BlockSpec(shape, lambda i: (0, 0))
    return pl.pallas_call(
        body,
        grid=(EP // BE,),
        in_specs=[
            pl.BlockSpec((GD, BE), lambda i: (0, i)),
            pl.BlockSpec((GD, BE), lambda i: (0, i)),
            pl.BlockSpec((16, BE), lambda i: (0, i)),
            full((32, GD)), full((32, GD)), full((32, 1)), full((32, 16)),
            full((32, 1)), full((32, 32)), full((32, 1)), full((32, 32)),
            full((32, 1)), full((1, 32)),
        ],
        out_specs=pl.BlockSpec((MD, BE), lambda i: (0, i)),
        out_shape=jax.ShapeDtypeStruct((MD, EP), jnp.float32),
    )(grT, gcT, eaT, at, bt, wrc, wet, b1c, e2t, b2c, c1t, c1bc, c2r)


def _tc_node(tab, a0, a1, n1h, n1m, n1b, n2, n2b, wo, bo):
    def body(tab_r, a0_r, a1_r, n1h_r, n1m_r, n1b_r, n2_r, n2b_r,
             wo_r, bo_r, tab_o, hp_o):
        h = tab_r[:, 0:32]
        x4 = tab_r[:, 32:36]
        magg = a0_r[:, 0:32] + a1_r[:, 0:32]
        tr = a0_r[:, 32:36] + a1_r[:, 32:36]
        cnt = a0_r[:, 36:37] + a1_r[:, 36:37]
        o = _silu(jnp.dot(h, n1h_r[:], preferred_element_type=jnp.float32)
                  + jnp.dot(magg, n1m_r[:], preferred_element_type=jnp.float32)
                  + n1b_r[:])
        hn = h + jnp.dot(o, n2_r[:],
                         preferred_element_type=jnp.float32) + n2b_r[:]
        xn = x4 + tr / jnp.clip(cnt, 1.0, None)
        tab_o[:] = jnp.concatenate([hn, xn], axis=1)
        hp_o[:] = jnp.dot(hn, wo_r[:],
                          preferred_element_type=jnp.float32) + bo_r[:]

    full = lambda shape: pl.BlockSpec(shape, lambda i: (0, 0))
    return pl.pallas_call(
        body,
        grid=(N // BN,),
        in_specs=[
            pl.BlockSpec((BN, GD), lambda i: (i, 0)),
            pl.BlockSpec((BN, MD), lambda i: (i, 0)),
            pl.BlockSpec((BN, MD), lambda i: (i, 0)),
            full((32, 32)), full((32, 32)), full((1, 32)),
            full((32, 32)), full((1, 32)), full((32, 32)), full((1, 32)),
        ],
        out_specs=(pl.BlockSpec((BN, GD), lambda i: (i, 0)),
                   pl.BlockSpec((BN, 32), lambda i: (i, 0))),
        out_shape=(jax.ShapeDtypeStruct((N, GD), jnp.float32),
                   jax.ShapeDtypeStruct((N, 32), jnp.float32)),
    )(tab, a0, a1, n1h, n1m, n1b, n2, n2b, wo, bo)


def kernel(h, x, edge_index, edge_attr, params):
    p = params
    row = edge_index[0]
    col = edge_index[1]
    row2 = jnp.pad(row, (0, EP - E)).reshape(NR, 128)
    col2 = jnp.pad(col, (0, EP - E)).reshape(NR, 128)
    eaT = jnp.pad(edge_attr, ((0, EP - E), (0, 0))).T
    zrows = jnp.zeros((ZR, MD), jnp.float32)
    eye = jnp.eye(32, dtype=jnp.float32)
    zb = jnp.zeros((1, 32), jnp.float32)
    z4 = jnp.zeros((32, 4), jnp.float32)

    tab = _tc_embed(h, x, p['emb_in_w'], p['emb_in_b'].reshape(1, 32))
    hproj = None

    for l in range(3):
        gr, gc = _sc_gather(tab, row2, col2)
        e1w = p[f'e1_w_{l}']
        mtT = _tc_edge(
            gr.T, gc.T, eaT,
            jnp.concatenate([e1w[0:32].T, z4], axis=1),
            jnp.concatenate([e1w[32:64].T, z4], axis=1),
            e1w[64:65].T, e1w[65:81].T,
            p[f'e1_b_{l}'].reshape(32, 1),
            p[f'e2_w_{l}'].T, p[f'e2_b_{l}'].reshape(32, 1),
            p[f'c1_w_{l}'].T, p[f'c1_b_{l}'].reshape(32, 1),
            p[f'c2_w_{l}'].reshape(1, 32))
        accp = _sc_scatter(mtT.T, row2, zrows)
        n1w = p[f'n1_w_{l}']
        if l < 2:
            wo, bo = eye, zb
        else:
            wo, bo = p['emb_out_w'], p['emb_out_b'].reshape(1, 32)
        tab, hproj = _tc_node(
            tab, accp[0], accp[1],
            n1w[0:32], n1w[32:64], p[f'n1_b_{l}'].reshape(1, 32),
            p[f'n2_w_{l}'], p[f'n2_b_{l}'].reshape(1, 32), wo, bo)

    return hproj, tab[:, 32:35]


# speedup vs baseline: 3.9048x; 1.0387x over previous
import functools

import jax
import jax.numpy as jnp
from jax import lax
from jax.experimental import pallas as pl
from jax.experimental.pallas import tpu as pltpu
from jax.experimental.pallas import tpu_sc as plsc

N = 50000
E = 800000
EP = 802816
NR = EP // 128
NC, NS = 2, 16
NW = NC * NS
RPW = NR // NW
G = 4
STEPS = RPW // G
GD = 40
MD = 40
NPS = N // NS
ZR = 125
BE = 4096
BN = 2000


def _mesh():
    return plsc.VectorSubcoreMesh(core_axis_name="c", subcore_axis_name="s")


_SC_PARAMS = functools.partial(
    pltpu.CompilerParams, use_tc_tiling_on_sc=False)


def _silu(v):
    return 0.5 * v * jnp.tanh(0.5 * v) + 0.5 * v


def _sc_gather(tab1, tab2, row2, col2):
    def body(t1_r, t2_r, row_r, col_r, gr_o, gc_o, rbuf, cbuf, grb, gcb, sem):
        wid = lax.axis_index("s") * NC + lax.axis_index("c")
        base = wid * RPW

        def step(g, carry):
            cid = base + g * G
            pltpu.sync_copy(row_r.at[pl.ds(cid, G)], rbuf)
            pltpu.sync_copy(col_r.at[pl.ds(cid, G)], cbuf)
            cps = []
            for j in range(G):
                cps.append(pltpu.async_copy(
                    t1_r.at[rbuf.at[j]], grb.at[pl.ds(j * 128, 128)], sem))
                cps.append(pltpu.async_copy(
                    t2_r.at[cbuf.at[j]], gcb.at[pl.ds(j * 128, 128)], sem))
            for cp in cps:
                cp.wait()
            e0 = cid * 128
            pltpu.sync_copy(grb, gr_o.at[pl.ds(e0, G * 128)])
            pltpu.sync_copy(gcb, gc_o.at[pl.ds(e0, G * 128)])
            return carry

        lax.fori_loop(0, STEPS, step, 0)

    return pl.kernel(
        body,
        out_type=(jax.ShapeDtypeStruct((EP, GD), jnp.float32),
                  jax.ShapeDtypeStruct((EP, GD), jnp.float32)),
        mesh=_mesh(),
        compiler_params=_SC_PARAMS(),
        scratch_types=[
            pltpu.VMEM((G, 128), jnp.int32),
            pltpu.VMEM((G, 128), jnp.int32),
            pltpu.VMEM((G * 128, GD), jnp.float32),
            pltpu.VMEM((G * 128, GD), jnp.float32),
            pltpu.SemaphoreType.DMA,
        ],
    )(tab1, tab2, row2, col2)


def _sc_scatter(mt, row2, zrows):
    def body(mt_r, row_r, z_r, out_r, acc, rbuf, mtb, sem):
        c = lax.axis_index("c")
        s = lax.axis_index("s")
        wid = s * NC + c
        r0 = s * NPS

        pltpu.sync_copy(z_r, mtb.at[pl.ds(0, ZR)])

        def zstep(k, carry):
            pltpu.sync_copy(mtb.at[pl.ds(0, ZR)],
                            acc.at[pl.ds(r0 + k * ZR, ZR)])
            return carry

        lax.fori_loop(0, NPS // ZR, zstep, 0)
        plsc.subcore_barrier()

        base = wid * RPW

        def step(g, carry):
            cid = base + g
            pltpu.sync_copy(row_r.at[pl.ds(cid, 1)], rbuf)
            pltpu.sync_copy(mt_r.at[pl.ds(cid * 128, 128)], mtb)
            pltpu.sync_copy(mtb, acc.at[rbuf.at[0]], add=True)
            return carry

        lax.fori_loop(0, RPW, step, 0)
        plsc.subcore_barrier()

        def ostep(k, carry):
            pltpu.sync_copy(acc.at[pl.ds(r0 + k * ZR, ZR)],
                            mtb.at[pl.ds(0, ZR)])
            pltpu.sync_copy(mtb.at[pl.ds(0, ZR)],
                            out_r.at[c, pl.ds(r0 + k * ZR, ZR)])
            return carry

        lax.fori_loop(0, NPS // ZR, ostep, 0)

    return pl.kernel(
        body,
        out_type=jax.ShapeDtypeStruct((NC, N, MD), jnp.float32),
        mesh=_mesh(),
        compiler_params=_SC_PARAMS(),
        scratch_types=[
            pltpu.VMEM_SHARED((N, MD), jnp.float32),
            pltpu.VMEM((1, 128), jnp.int32),
            pltpu.VMEM((128, MD), jnp.float32),
            pltpu.SemaphoreType.DMA,
        ],
    )(mt, row2, zrows)


def _tc_embed(h, x, w, b, ws, wt):
    def body(h_r, x_r, w_r, b_r, ws_r, wt_r, tab_o, t1_o, t2_o):
        h0 = jnp.dot(h_r[:], w_r[:],
                     preferred_element_type=jnp.float32) + b_r[:]
        xv = x_r[:]
        z1 = jnp.zeros((BN, 1), jnp.float32)
        z5 = jnp.zeros((BN, 5), jnp.float32)
        tab_o[:] = jnp.concatenate([h0, xv, z1], axis=1)
        t1_o[:] = jnp.concatenate(
            [jnp.dot(h0, ws_r[:], preferred_element_type=jnp.float32),
             xv, z5], axis=1)
        t2_o[:] = jnp.concatenate(
            [jnp.dot(h0, wt_r[:], preferred_element_type=jnp.float32),
             -xv, z5], axis=1)

    return pl.pallas_call(
        body,
        grid=(N // BN,),
        in_specs=[
            pl.BlockSpec((BN, 125), lambda i: (i, 0)),
            pl.BlockSpec((BN, 3), lambda i: (i, 0)),
            pl.BlockSpec((125, 32), lambda i: (0, 0)),
            pl.BlockSpec((1, 32), lambda i: (0, 0)),
            pl.BlockSpec((32, 32), lambda i: (0, 0)),
            pl.BlockSpec((32, 32), lambda i: (0, 0)),
        ],
        out_specs=(pl.BlockSpec((BN, 36), lambda i: (i, 0)),
                   pl.BlockSpec((BN, GD), lambda i: (i, 0)),
                   pl.BlockSpec((BN, GD), lambda i: (i, 0))),
        out_shape=(jax.ShapeDtypeStruct((N, 36), jnp.float32),
                   jax.ShapeDtypeStruct((N, GD), jnp.float32),
                   jax.ShapeDtypeStruct((N, GD), jnp.float32)),
    )(h, x, w, b, ws, wt)


def _tc_edge(gsT, eaT, wrc, wet, b1c, e2t, b2c, c1t, c1bc, c2r):
    def body(gs_r, ea_r, wrc_r, wet_r, b1c_r, e2t_r,
             b2c_r, c1t_r, c1bc_r, c2r_r, mt_o):
        pid = pl.program_id(0)
        g = gs_r[:]
        d = g[32:36, :]
        radial = jnp.sum(d * d, axis=0, keepdims=True)
        pre = (g[0:32, :]
               + jnp.dot(wet_r[:], ea_r[:],
                         preferred_element_type=jnp.float32)
               + wrc_r[:] * radial + b1c_r[:])
        m1 = _silu(pre)
        m = _silu(jnp.dot(e2t_r[:], m1,
                          preferred_element_type=jnp.float32) + b2c_r[:])
        t = _silu(jnp.dot(c1t_r[:], m,
                          preferred_element_type=jnp.float32) + c1bc_r[:])
        sc = jnp.dot(c2r_r[:], t, preferred_element_type=jnp.float32)
        ids = pid * BE + lax.broadcasted_iota(jnp.int32, (1, BE), 1)
        validf = (ids < E).astype(jnp.float32)
        mt = jnp.concatenate(
            [m, d * sc, jnp.ones((1, BE), jnp.float32),
             jnp.zeros((3, BE), jnp.float32)], axis=0) * validf
        mt_o[:] = mt

    full = lambda shape: pl.BlockSpec(shape, lambda i: (0, 0))
    return pl.pallas_call(
        body,
        grid=(EP // BE,),
        in_specs=[
            pl.BlockSpec((GD, BE), lambda i: (0, i)),
            pl.BlockSpec((16, BE), lambda i: (0, i)),
            full((32, 1)), full((32, 16)), full((32, 1)), full((32, 32)),
            full((32, 1)), full((32, 32)), full((32, 1)), full((1, 32)),
        ],
        out_specs=pl.BlockSpec((MD, BE), lambda i: (0, i)),
        out_shape=jax.ShapeDtypeStruct((MD, EP), jnp.float32),
    )(gsT, eaT, wrc, wet, b1c, e2t, b2c, c1t, c1bc, c2r)


def _tc_node(tab, a0, a1, n1h, n1m, n1b, n2, n2b, wo, bo, wsn, wtn):
    def body(tab_r, a0_r, a1_r, n1h_r, n1m_r, n1b_r, n2_r, n2b_r,
             wo_r, bo_r, wsn_r, wtn_r, tab_o, t1_o, t2_o, hp_o):
        h = tab_r[:, 0:32]
        x4 = tab_r[:, 32:36]
        magg = a0_r[:, 0:32] + a1_r[:, 0:32]
        tr = a0_r[:, 32:36] + a1_r[:, 32:36]
        cnt = a0_r[:, 36:37] + a1_r[:, 36:37]
        o = _silu(jnp.dot(h, n1h_r[:], preferred_element_type=jnp.float32)
                  + jnp.dot(magg, n1m_r[:], preferred_element_type=jnp.float32)
                  + n1b_r[:])
        hn = h + jnp.dot(o, n2_r[:],
                         preferred_element_type=jnp.float32) + n2b_r[:]
        xn = x4 + tr / jnp.clip(cnt, 1.0, None)
        xn3 = xn[:, 0:3]
        z5 = jnp.zeros((BN, 5), jnp.float32)
        tab_o[:] = jnp.concatenate([hn, xn], axis=1)
        t1_o[:] = jnp.concatenate(
            [jnp.dot(hn, wsn_r[:], preferred_element_type=jnp.float32),
             xn3, z5], axis=1)
        t2_o[:] = jnp.concatenate(
            [jnp.dot(hn, wtn_r[:], preferred_element_type=jnp.float32),
             -xn3, z5], axis=1)
        hp_o[:] = jnp.dot(hn, wo_r[:],
                          preferred_element_type=jnp.float32) + bo_r[:]

    full = lambda shape: pl.BlockSpec(shape, lambda i: (0, 0))
    return pl.pallas_call(
        body,
        grid=(N // BN,),
        in_specs=[
            pl.BlockSpec((BN, 36), lambda i: (i, 0)),
            pl.BlockSpec((BN, MD), lambda i: (i, 0)),
            pl.BlockSpec((BN, MD), lambda i: (i, 0)),
            full((32, 32)), full((32, 32)), full((1, 32)),
            full((32, 32)), full((1, 32)), full((32, 32)), full((1, 32)),
            full((32, 32)), full((32, 32)),
        ],
        out_specs=(pl.BlockSpec((BN, 36), lambda i: (i, 0)),
                   pl.BlockSpec((BN, GD), lambda i: (i, 0)),
                   pl.BlockSpec((BN, GD), lambda i: (i, 0)),
                   pl.BlockSpec((BN, 32), lambda i: (i, 0))),
        out_shape=(jax.ShapeDtypeStruct((N, 36), jnp.float32),
                   jax.ShapeDtypeStruct((N, GD), jnp.float32),
                   jax.ShapeDtypeStruct((N, GD), jnp.float32),
                   jax.ShapeDtypeStruct((N, 32), jnp.float32)),
    )(tab, a0, a1, n1h, n1m, n1b, n2, n2b, wo, bo, wsn, wtn)


def kernel(h, x, edge_index, edge_attr, params):
    p = params
    row = edge_index[0]
    col = edge_index[1]
    row2 = jnp.pad(row, (0, EP - E)).reshape(NR, 128)
    col2 = jnp.pad(col, (0, EP - E)).reshape(NR, 128)
    eaT = jnp.pad(edge_attr, ((0, EP - E), (0, 0))).T
    zrows = jnp.zeros((ZR, MD), jnp.float32)
    eye = jnp.eye(32, dtype=jnp.float32)
    zb = jnp.zeros((1, 32), jnp.float32)
    zw = jnp.zeros((32, 32), jnp.float32)

    e1w0 = p['e1_w_0']
    tab, t1, t2 = _tc_embed(h, x, p['emb_in_w'], p['emb_in_b'].reshape(1, 32),
                            e1w0[0:32], e1w0[32:64])
    hproj = None

    for l in range(3):
        gr, gc = _sc_gather(t1, t2, row2, col2)
        gsT = (gr + gc).T
        e1w = p[f'e1_w_{l}']
        mtT = _tc_edge(
            gsT, eaT,
            e1w[64:65].T, e1w[65:81].T,
            p[f'e1_b_{l}'].reshape(32, 1),
            p[f'e2_w_{l}'].T, p[f'e2_b_{l}'].reshape(32, 1),
            p[f'c1_w_{l}'].T, p[f'c1_b_{l}'].reshape(32, 1),
            p[f'c2_w_{l}'].reshape(1, 32))
        accp = _sc_scatter(mtT.T, row2, zrows)
        n1w = p[f'n1_w_{l}']
        if l < 2:
            wo, bo = eye, zb
            e1wn = p[f'e1_w_{l + 1}']
            wsn, wtn = e1wn[0:32], e1wn[32:64]
        else:
            wo, bo = p['emb_out_w'], p['emb_out_b'].reshape(1, 32)
            wsn, wtn = zw, zw
        tab, t1, t2, hproj = _tc_node(
            tab, accp[0], accp[1],
            n1w[0:32], n1w[32:64], p[f'n1_b_{l}'].reshape(1, 32),
            p[f'n2_w_{l}'], p[f'n2_b_{l}'].reshape(1, 32), wo, bo, wsn, wtn)

    return hproj, tab[:, 32:35]


# speedup vs baseline: 4.0758x; 1.0438x over previous
import functools

import jax
import jax.numpy as jnp
from jax import lax
from jax.experimental import pallas as pl
from jax.experimental.pallas import tpu as pltpu
from jax.experimental.pallas import tpu_sc as plsc

N = 50000
E = 800000
EP = 802816
NR = EP // 128
NC, NS = 2, 16
NW = NC * NS
RPW = NR // NW
G = 4
STEPS = RPW // G
GD = 40
MD = 40
NPS = N // NS
ZR = 125
BE = 4096
BN = 2000


def _mesh():
    return plsc.VectorSubcoreMesh(core_axis_name="c", subcore_axis_name="s")


_SC_PARAMS = functools.partial(
    pltpu.CompilerParams, use_tc_tiling_on_sc=False)


def _silu(v):
    return 0.5 * v * jnp.tanh(0.5 * v) + 0.5 * v


def _sc_gather(tab1, tab2, row2, col2):
    def body(t1_r, t2_r, row_r, col_r, gr_o, gc_o, rbuf, cbuf, grb, gcb, sem):
        wid = lax.axis_index("s") * NC + lax.axis_index("c")
        base = wid * RPW

        def step(g, carry):
            cid = base + g * G
            pltpu.sync_copy(row_r.at[pl.ds(cid, G)], rbuf)
            pltpu.sync_copy(col_r.at[pl.ds(cid, G)], cbuf)
            cps = []
            for j in range(G):
                cps.append(pltpu.async_copy(
                    t1_r.at[rbuf.at[j]], grb.at[pl.ds(j * 128, 128)], sem))
                cps.append(pltpu.async_copy(
                    t2_r.at[cbuf.at[j]], gcb.at[pl.ds(j * 128, 128)], sem))
            for cp in cps:
                cp.wait()
            e0 = cid * 128
            pltpu.sync_copy(grb, gr_o.at[pl.ds(e0, G * 128)])
            pltpu.sync_copy(gcb, gc_o.at[pl.ds(e0, G * 128)])
            return carry

        lax.fori_loop(0, STEPS, step, 0)

    return pl.kernel(
        body,
        out_type=(jax.ShapeDtypeStruct((EP, GD), jnp.float32),
                  jax.ShapeDtypeStruct((EP, GD), jnp.float32)),
        mesh=_mesh(),
        compiler_params=_SC_PARAMS(),
        scratch_types=[
            pltpu.VMEM((G, 128), jnp.int32),
            pltpu.VMEM((G, 128), jnp.int32),
            pltpu.VMEM((G * 128, GD), jnp.float32),
            pltpu.VMEM((G * 128, GD), jnp.float32),
            pltpu.SemaphoreType.DMA,
        ],
    )(tab1, tab2, row2, col2)


def _sc_scatter(mt, row2, zrows):
    def body(mt_r, row_r, z_r, out_r, acc, rbuf, mtb, sem):
        c = lax.axis_index("c")
        s = lax.axis_index("s")
        wid = s * NC + c
        r0 = s * NPS

        pltpu.sync_copy(z_r, mtb.at[pl.ds(0, ZR)])

        def zstep(k, carry):
            pltpu.sync_copy(mtb.at[pl.ds(0, ZR)],
                            acc.at[pl.ds(r0 + k * ZR, ZR)])
            return carry

        lax.fori_loop(0, NPS // ZR, zstep, 0)
        plsc.subcore_barrier()

        base = wid * RPW

        def step(g, carry):
            cid = base + g
            pltpu.sync_copy(row_r.at[pl.ds(cid, 1)], rbuf)
            pltpu.sync_copy(mt_r.at[pl.ds(cid * 128, 128)], mtb)
            pltpu.sync_copy(mtb, acc.at[rbuf.at[0]], add=True)
            return carry

        lax.fori_loop(0, RPW, step, 0)
        plsc.subcore_barrier()

        def ostep(k, carry):
            pltpu.sync_copy(acc.at[pl.ds(r0 + k * ZR, ZR)],
                            mtb.at[pl.ds(0, ZR)])
            pltpu.sync_copy(mtb.at[pl.ds(0, ZR)],
                            out_r.at[c, pl.ds(r0 + k * ZR, ZR)])
            return carry

        lax.fori_loop(0, NPS // ZR, ostep, 0)

    return pl.kernel(
        body,
        out_type=jax.ShapeDtypeStruct((NC, N, MD), jnp.float32),
        mesh=_mesh(),
        compiler_params=_SC_PARAMS(),
        scratch_types=[
            pltpu.VMEM_SHARED((N, MD), jnp.float32),
            pltpu.VMEM((1, 128), jnp.int32),
            pltpu.VMEM((128, MD), jnp.float32),
            pltpu.SemaphoreType.DMA,
        ],
    )(mt, row2, zrows)


def _tc_embed(h, x, w, b, ws, wt):
    def body(h_r, x_r, w_r, b_r, ws_r, wt_r, tab_o, t1_o, t2_o):
        h0 = jnp.dot(h_r[:], w_r[:],
                     preferred_element_type=jnp.float32) + b_r[:]
        xv = x_r[:]
        z1 = jnp.zeros((BN, 1), jnp.float32)
        z5 = jnp.zeros((BN, 5), jnp.float32)
        tab_o[:] = jnp.concatenate([h0, xv, z1], axis=1)
        t1_o[:] = jnp.concatenate(
            [jnp.dot(h0, ws_r[:], preferred_element_type=jnp.float32),
             xv, z5], axis=1)
        t2_o[:] = jnp.concatenate(
            [jnp.dot(h0, wt_r[:], preferred_element_type=jnp.float32),
             -xv, z5], axis=1)

    return pl.pallas_call(
        body,
        grid=(N // BN,),
        in_specs=[
            pl.BlockSpec((BN, 125), lambda i: (i, 0)),
            pl.BlockSpec((BN, 3), lambda i: (i, 0)),
            pl.BlockSpec((125, 32), lambda i: (0, 0)),
            pl.BlockSpec((1, 32), lambda i: (0, 0)),
            pl.BlockSpec((32, 32), lambda i: (0, 0)),
            pl.BlockSpec((32, 32), lambda i: (0, 0)),
        ],
        out_specs=(pl.BlockSpec((BN, 36), lambda i: (i, 0)),
                   pl.BlockSpec((BN, GD), lambda i: (i, 0)),
                   pl.BlockSpec((BN, GD), lambda i: (i, 0))),
        out_shape=(jax.ShapeDtypeStruct((N, 36), jnp.float32),
                   jax.ShapeDtypeStruct((N, GD), jnp.float32),
                   jax.ShapeDtypeStruct((N, GD), jnp.float32)),
    )(h, x, w, b, ws, wt)


def _tc_edge(gr, gc, ea, r4, we, b1, e2, b2, c1, c1b, c2):
    def body(gr_r, gc_r, ea_r, r4_r, we_r, b1_r, e2_r,
             b2_r, c1_r, c1b_r, c2_r, mt_o):
        pid = pl.program_id(0)
        g = gr_r[:] + gc_r[:]
        d = g[:, 32:36]
        dsq = d * d
        pre = (g[:, 0:32]
               + jnp.dot(ea_r[:], we_r[:], preferred_element_type=jnp.float32)
               + jnp.dot(dsq, r4_r[:], preferred_element_type=jnp.float32)
               + b1_r[:])
        m1 = _silu(pre)
        m = _silu(jnp.dot(m1, e2_r[:],
                          preferred_element_type=jnp.float32) + b2_r[:])
        t = _silu(jnp.dot(m, c1_r[:],
                          preferred_element_type=jnp.float32) + c1b_r[:])
        sc = jnp.dot(t, c2_r[:], preferred_element_type=jnp.float32)
        ids = pid * BE + lax.broadcasted_iota(jnp.int32, (BE, 1), 0)
        validf = (ids < E).astype(jnp.float32)
        mt = jnp.concatenate(
            [m, d * sc, jnp.ones((BE, 1), jnp.float32),
             jnp.zeros((BE, 3), jnp.float32)], axis=1) * validf
        mt_o[:] = mt

    full = lambda shape: pl.BlockSpec(shape, lambda i: (0, 0))
    return pl.pallas_call(
        body,
        grid=(EP // BE,),
        in_specs=[
            pl.BlockSpec((BE, GD), lambda i: (i, 0)),
            pl.BlockSpec((BE, GD), lambda i: (i, 0)),
            pl.BlockSpec((BE, 16), lambda i: (i, 0)),
            full((4, 32)), full((16, 32)), full((1, 32)), full((32, 32)),
            full((1, 32)), full((32, 32)), full((1, 32)), full((32, 1)),
        ],
        out_specs=pl.BlockSpec((BE, MD), lambda i: (i, 0)),
        out_shape=jax.ShapeDtypeStruct((EP, MD), jnp.float32),
    )(gr, gc, ea, r4, we, b1, e2, b2, c1, c1b, c2)


def _tc_node(tab, a0, a1, n1h, n1m, n1b, n2, n2b, wo, bo, wsn, wtn):
    def body(tab_r, a0_r, a1_r, n1h_r, n1m_r, n1b_r, n2_r, n2b_r,
             wo_r, bo_r, wsn_r, wtn_r, tab_o, t1_o, t2_o, hp_o):
        h = tab_r[:, 0:32]
        x4 = tab_r[:, 32:36]
        magg = a0_r[:, 0:32] + a1_r[:, 0:32]
        tr = a0_r[:, 32:36] + a1_r[:, 32:36]
        cnt = a0_r[:, 36:37] + a1_r[:, 36:37]
        o = _silu(jnp.dot(h, n1h_r[:], preferred_element_type=jnp.float32)
                  + jnp.dot(magg, n1m_r[:], preferred_element_type=jnp.float32)
                  + n1b_r[:])
        hn = h + jnp.dot(o, n2_r[:],
                         preferred_element_type=jnp.float32) + n2b_r[:]
        xn = x4 + tr / jnp.clip(cnt, 1.0, None)
        xn3 = xn[:, 0:3]
        z5 = jnp.zeros((BN, 5), jnp.float32)
        tab_o[:] = jnp.concatenate([hn, xn], axis=1)
        t1_o[:] = jnp.concatenate(
            [jnp.dot(hn, wsn_r[:], preferred_element_type=jnp.float32),
             xn3, z5], axis=1)
        t2_o[:] = jnp.concatenate(
            [jnp.dot(hn, wtn_r[:], preferred_element_type=jnp.float32),
             -xn3, z5], axis=1)
        hp_o[:] = jnp.dot(hn, wo_r[:],
                          preferred_element_type=jnp.float32) + bo_r[:]

    full = lambda shape: pl.BlockSpec(shape, lambda i: (0, 0))
    return pl.pallas_call(
        body,
        grid=(N // BN,),
        in_specs=[
            pl.BlockSpec((BN, 36), lambda i: (i, 0)),
            pl.BlockSpec((BN, MD), lambda i: (i, 0)),
            pl.BlockSpec((BN, MD), lambda i: (i, 0)),
            full((32, 32)), full((32, 32)), full((1, 32)),
            full((32, 32)), full((1, 32)), full((32, 32)), full((1, 32)),
            full((32, 32)), full((32, 32)),
        ],
        out_specs=(pl.BlockSpec((BN, 36), lambda i: (i, 0)),
                   pl.BlockSpec((BN, GD), lambda i: (i, 0)),
                   pl.BlockSpec((BN, GD), lambda i: (i, 0)),
                   pl.BlockSpec((BN, 32), lambda i: (i, 0))),
        out_shape=(jax.ShapeDtypeStruct((N, 36), jnp.float32),
                   jax.ShapeDtypeStruct((N, GD), jnp.float32),
                   jax.ShapeDtypeStruct((N, GD), jnp.float32),
                   jax.ShapeDtypeStruct((N, 32), jnp.float32)),
    )(tab, a0, a1, n1h, n1m, n1b, n2, n2b, wo, bo, wsn, wtn)


def kernel(h, x, edge_index, edge_attr, params):
    p = params
    row = edge_index[0]
    col = edge_index[1]
    row2 = jnp.pad(row, (0, EP - E)).reshape(NR, 128)
    col2 = jnp.pad(col, (0, EP - E)).reshape(NR, 128)
    eap = jnp.pad(edge_attr, ((0, EP - E), (0, 0)))
    zrows = jnp.zeros((ZR, MD), jnp.float32)
    eye = jnp.eye(32, dtype=jnp.float32)
    zb = jnp.zeros((1, 32), jnp.float32)
    zw = jnp.zeros((32, 32), jnp.float32)

    e1w0 = p['e1_w_0']
    tab, t1, t2 = _tc_embed(h, x, p['emb_in_w'], p['emb_in_b'].reshape(1, 32),
                            e1w0[0:32], e1w0[32:64])
    hproj = None

    for l in range(3):
        gr, gc = _sc_gather(t1, t2, row2, col2)
        e1w = p[f'e1_w_{l}']
        wr = e1w[64:65]
        r4 = jnp.concatenate([wr, wr, wr, zb], axis=0)
        mt = _tc_edge(
            gr, gc, eap,
            r4, e1w[65:81],
            p[f'e1_b_{l}'].reshape(1, 32),
            p[f'e2_w_{l}'], p[f'e2_b_{l}'].reshape(1, 32),
            p[f'c1_w_{l}'], p[f'c1_b_{l}'].reshape(1, 32),
            p[f'c2_w_{l}'])
        accp = _sc_scatter(mt, row2, zrows)
        n1w = p[f'n1_w_{l}']
        if l < 2:
            wo, bo = eye, zb
            e1wn = p[f'e1_w_{l + 1}']
            wsn, wtn = e1wn[0:32], e1wn[32:64]
        else:
            wo, bo = p['emb_out_w'], p['emb_out_b'].reshape(1, 32)
            wsn, wtn = zw, zw
        tab, t1, t2, hproj = _tc_node(
            tab, accp[0], accp[1],
            n1w[0:32], n1w[32:64], p[f'n1_b_{l}'].reshape(1, 32),
            p[f'n2_w_{l}'], p[f'n2_b_{l}'].reshape(1, 32), wo, bo, wsn, wtn)

    return hproj, tab[:, 32:35]


# speedup vs baseline: 5.0056x; 1.2281x over previous
import functools

import jax
import jax.numpy as jnp
from jax import lax
from jax.experimental import pallas as pl
from jax.experimental.pallas import tpu as pltpu
from jax.experimental.pallas import tpu_sc as plsc

N = 50000
E = 800000
EP = 802816
NR = EP // 128
NC, NS = 2, 16
NW = NC * NS
RPW = NR // NW
G = 4
STEPS = RPW // G
GD = 48
MD = 40
NPS = N // NS
ZR = 125
BE = 4096
BN = 2000


def _mesh():
    return plsc.VectorSubcoreMesh(core_axis_name="c", subcore_axis_name="s")


_SC_PARAMS = functools.partial(
    pltpu.CompilerParams, use_tc_tiling_on_sc=False)


def _silu(v):
    return 0.5 * v * jnp.tanh(0.5 * v) + 0.5 * v


def _sc_gather(tab1, tab2, row2, col2):
    CE = G * 128

    def body(t1_r, t2_r, row_r, col_r, gs_o,
             rbuf0, cbuf0, rbuf1, cbuf1, grb0, gcb0, grb1, gcb1, sem0, sem1):
        wid = lax.axis_index("s") * NC + lax.axis_index("c")
        base = wid * RPW

        def fire(g, rbuf, cbuf, grb, gcb, sem):
            cid = base + g * G
            pltpu.sync_copy(row_r.at[pl.ds(cid, G)], rbuf)
            pltpu.sync_copy(col_r.at[pl.ds(cid, G)], cbuf)
            for j in range(G):
                pltpu.async_copy(
                    t1_r.at[rbuf.at[j]], grb.at[pl.ds(j * 128, 128)], sem)
                pltpu.async_copy(
                    t2_r.at[cbuf.at[j]], gcb.at[pl.ds(j * 128, 128)], sem)

        def drain(rbuf, cbuf, grb, gcb, sem):
            for j in range(G):
                pltpu.make_async_copy(
                    t1_r.at[rbuf.at[j]], grb.at[pl.ds(j * 128, 128)],
                    sem).wait()
                pltpu.make_async_copy(
                    t2_r.at[cbuf.at[j]], gcb.at[pl.ds(j * 128, 128)],
                    sem).wait()

        def addstore(g, grb, gcb):
            def add4(i, carry):
                for u in range(4):
                    for k in range(3):
                        sl = pl.ds(k * 16, 16)
                        grb[i * 4 + u, sl] = grb[i * 4 + u, sl] + \
                            gcb[i * 4 + u, sl]
                return carry

            lax.fori_loop(0, CE // 4, add4, 0)
            cid = base + g * G
            pltpu.sync_copy(grb, gs_o.at[pl.ds(cid * 128, CE)])

        fire(0, rbuf0, cbuf0, grb0, gcb0, sem0)

        def step(k, carry):
            g0 = k * 2
            drain(rbuf0, cbuf0, grb0, gcb0, sem0)
            fire(g0 + 1, rbuf1, cbuf1, grb1, gcb1, sem1)
            addstore(g0, grb0, gcb0)
            drain(rbuf1, cbuf1, grb1, gcb1, sem1)
            fire(g0 + 2, rbuf0, cbuf0, grb0, gcb0, sem0)
            addstore(g0 + 1, grb1, gcb1)
            return carry

        lax.fori_loop(0, (STEPS - 1) // 2, step, 0)
        drain(rbuf0, cbuf0, grb0, gcb0, sem0)
        addstore(STEPS - 1, grb0, gcb0)

    return pl.kernel(
        body,
        out_type=jax.ShapeDtypeStruct((EP, GD), jnp.float32),
        mesh=_mesh(),
        compiler_params=_SC_PARAMS(),
        scratch_types=[
            pltpu.VMEM((G, 128), jnp.int32),
            pltpu.VMEM((G, 128), jnp.int32),
            pltpu.VMEM((G, 128), jnp.int32),
            pltpu.VMEM((G, 128), jnp.int32),
            pltpu.VMEM((CE, GD), jnp.float32),
            pltpu.VMEM((CE, GD), jnp.float32),
            pltpu.VMEM((CE, GD), jnp.float32),
            pltpu.VMEM((CE, GD), jnp.float32),
            pltpu.SemaphoreType.DMA,
            pltpu.SemaphoreType.DMA,
        ],
    )(tab1, tab2, row2, col2)


def _sc_scatter(mt, row2, zrows):
    def body(mt_r, row_r, z_r, out_r, acc, rbuf, mtb, sem):
        c = lax.axis_index("c")
        s = lax.axis_index("s")
        wid = s * NC + c
        r0 = s * NPS

        pltpu.sync_copy(z_r, mtb.at[pl.ds(0, ZR)])

        def zstep(k, carry):
            pltpu.sync_copy(mtb.at[pl.ds(0, ZR)],
                            acc.at[pl.ds(r0 + k * ZR, ZR)])
            return carry

        lax.fori_loop(0, NPS // ZR, zstep, 0)
        plsc.subcore_barrier()

        base = wid * RPW

        def step(g, carry):
            cid = base + g
            pltpu.sync_copy(row_r.at[pl.ds(cid, 1)], rbuf)
            pltpu.sync_copy(mt_r.at[pl.ds(cid * 128, 128)], mtb)
            pltpu.sync_copy(mtb, acc.at[rbuf.at[0]], add=True)
            return carry

        lax.fori_loop(0, RPW, step, 0)
        plsc.subcore_barrier()

        def ostep(k, carry):
            pltpu.sync_copy(acc.at[pl.ds(r0 + k * ZR, ZR)],
                            mtb.at[pl.ds(0, ZR)])
            pltpu.sync_copy(mtb.at[pl.ds(0, ZR)],
                            out_r.at[c, pl.ds(r0 + k * ZR, ZR)])
            return carry

        lax.fori_loop(0, NPS // ZR, ostep, 0)

    return pl.kernel(
        body,
        out_type=jax.ShapeDtypeStruct((NC, N, MD), jnp.float32),
        mesh=_mesh(),
        compiler_params=_SC_PARAMS(),
        scratch_types=[
            pltpu.VMEM_SHARED((N, MD), jnp.float32),
            pltpu.VMEM((1, 128), jnp.int32),
            pltpu.VMEM((128, MD), jnp.float32),
            pltpu.SemaphoreType.DMA,
        ],
    )(mt, row2, zrows)


def _tc_embed(h, x, w, b, ws, wt):
    def body(h_r, x_r, w_r, b_r, ws_r, wt_r, tab_o, t1_o, t2_o):
        h0 = jnp.dot(h_r[:], w_r[:],
                     preferred_element_type=jnp.float32) + b_r[:]
        xv = x_r[:]
        z1 = jnp.zeros((BN, 1), jnp.float32)
        z13 = jnp.zeros((BN, 13), jnp.float32)
        tab_o[:] = jnp.concatenate([h0, xv, z1], axis=1)
        t1_o[:] = jnp.concatenate(
            [jnp.dot(h0, ws_r[:], preferred_element_type=jnp.float32),
             xv, z13], axis=1)
        t2_o[:] = jnp.concatenate(
            [jnp.dot(h0, wt_r[:], preferred_element_type=jnp.float32),
             -xv, z13], axis=1)

    return pl.pallas_call(
        body,
        grid=(N // BN,),
        in_specs=[
            pl.BlockSpec((BN, 125), lambda i: (i, 0)),
            pl.BlockSpec((BN, 3), lambda i: (i, 0)),
            pl.BlockSpec((125, 32), lambda i: (0, 0)),
            pl.BlockSpec((1, 32), lambda i: (0, 0)),
            pl.BlockSpec((32, 32), lambda i: (0, 0)),
            pl.BlockSpec((32, 32), lambda i: (0, 0)),
        ],
        out_specs=(pl.BlockSpec((BN, 36), lambda i: (i, 0)),
                   pl.BlockSpec((BN, GD), lambda i: (i, 0)),
                   pl.BlockSpec((BN, GD), lambda i: (i, 0))),
        out_shape=(jax.ShapeDtypeStruct((N, 36), jnp.float32),
                   jax.ShapeDtypeStruct((N, GD), jnp.float32),
                   jax.ShapeDtypeStruct((N, GD), jnp.float32)),
    )(h, x, w, b, ws, wt)


def _tc_edge(gs, ea, r4, we, b1, e2, b2, c1, c1b, c2):
    def body(gs_r, ea_r, r4_r, we_r, b1_r, e2_r,
             b2_r, c1_r, c1b_r, c2_r, mt_o):
        pid = pl.program_id(0)
        g = gs_r[:]
        d = g[:, 32:36]
        dsq = d * d
        pre = (g[:, 0:32]
               + jnp.dot(ea_r[:], we_r[:], preferred_element_type=jnp.float32)
               + jnp.dot(dsq, r4_r[:], preferred_element_type=jnp.float32)
               + b1_r[:])
        m1 = _silu(pre)
        m = _silu(jnp.dot(m1, e2_r[:],
                          preferred_element_type=jnp.float32) + b2_r[:])
        t = _silu(jnp.dot(m, c1_r[:],
                          preferred_element_type=jnp.float32) + c1b_r[:])
        sc = jnp.dot(t, c2_r[:], preferred_element_type=jnp.float32)
        ids = pid * BE + lax.broadcasted_iota(jnp.int32, (BE, 1), 0)
        validf = (ids < E).astype(jnp.float32)
        mt = jnp.concatenate(
            [m, d * sc, jnp.ones((BE, 1), jnp.float32),
             jnp.zeros((BE, 3), jnp.float32)], axis=1) * validf
        mt_o[:] = mt

    full = lambda shape: pl.BlockSpec(shape, lambda i: (0, 0))
    return pl.pallas_call(
        body,
        grid=(EP // BE,),
        in_specs=[
            pl.BlockSpec((BE, GD), lambda i: (i, 0)),
            pl.BlockSpec((BE, 16), lambda i: (i, 0)),
            full((4, 32)), full((16, 32)), full((1, 32)), full((32, 32)),
            full((1, 32)), full((32, 32)), full((1, 32)), full((32, 1)),
        ],
        out_specs=pl.BlockSpec((BE, MD), lambda i: (i, 0)),
        out_shape=jax.ShapeDtypeStruct((EP, MD), jnp.float32),
    )(gs, ea, r4, we, b1, e2, b2, c1, c1b, c2)


def _tc_node(tab, a0, a1, n1h, n1m, n1b, n2, n2b, wo, bo, wsn, wtn):
    def body(tab_r, a0_r, a1_r, n1h_r, n1m_r, n1b_r, n2_r, n2b_r,
             wo_r, bo_r, wsn_r, wtn_r, tab_o, t1_o, t2_o, hp_o):
        h = tab_r[:, 0:32]
        x4 = tab_r[:, 32:36]
        magg = a0_r[:, 0:32] + a1_r[:, 0:32]
        tr = a0_r[:, 32:36] + a1_r[:, 32:36]
        cnt = a0_r[:, 36:37] + a1_r[:, 36:37]
        o = _silu(jnp.dot(h, n1h_r[:], preferred_element_type=jnp.float32)
                  + jnp.dot(magg, n1m_r[:], preferred_element_type=jnp.float32)
                  + n1b_r[:])
        hn = h + jnp.dot(o, n2_r[:],
                         preferred_element_type=jnp.float32) + n2b_r[:]
        xn = x4 + tr / jnp.clip(cnt, 1.0, None)
        xn3 = xn[:, 0:3]
        z13 = jnp.zeros((BN, 13), jnp.float32)
        tab_o[:] = jnp.concatenate([hn, xn], axis=1)
        t1_o[:] = jnp.concatenate(
            [jnp.dot(hn, wsn_r[:], preferred_element_type=jnp.float32),
             xn3, z13], axis=1)
        t2_o[:] = jnp.concatenate(
            [jnp.dot(hn, wtn_r[:], preferred_element_type=jnp.float32),
             -xn3, z13], axis=1)
        hp_o[:] = jnp.dot(hn, wo_r[:],
                          preferred_element_type=jnp.float32) + bo_r[:]

    full = lambda shape: pl.BlockSpec(shape, lambda i: (0, 0))
    return pl.pallas_call(
        body,
        grid=(N // BN,),
        in_specs=[
            pl.BlockSpec((BN, 36), lambda i: (i, 0)),
            pl.BlockSpec((BN, MD), lambda i: (i, 0)),
            pl.BlockSpec((BN, MD), lambda i: (i, 0)),
            full((32, 32)), full((32, 32)), full((1, 32)),
            full((32, 32)), full((1, 32)), full((32, 32)), full((1, 32)),
            full((32, 32)), full((32, 32)),
        ],
        out_specs=(pl.BlockSpec((BN, 36), lambda i: (i, 0)),
                   pl.BlockSpec((BN, GD), lambda i: (i, 0)),
                   pl.BlockSpec((BN, GD), lambda i: (i, 0)),
                   pl.BlockSpec((BN, 32), lambda i: (i, 0))),
        out_shape=(jax.ShapeDtypeStruct((N, 36), jnp.float32),
                   jax.ShapeDtypeStruct((N, GD), jnp.float32),
                   jax.ShapeDtypeStruct((N, GD), jnp.float32),
                   jax.ShapeDtypeStruct((N, 32), jnp.float32)),
    )(tab, a0, a1, n1h, n1m, n1b, n2, n2b, wo, bo, wsn, wtn)


def kernel(h, x, edge_index, edge_attr, params):
    p = params
    row = edge_index[0]
    col = edge_index[1]
    row2 = jnp.pad(row, (0, EP - E)).reshape(NR, 128)
    col2 = jnp.pad(col, (0, EP - E)).reshape(NR, 128)
    eap = jnp.pad(edge_attr, ((0, EP - E), (0, 0)))
    zrows = jnp.zeros((ZR, MD), jnp.float32)
    eye = jnp.eye(32, dtype=jnp.float32)
    zb = jnp.zeros((1, 32), jnp.float32)
    zw = jnp.zeros((32, 32), jnp.float32)

    e1w0 = p['e1_w_0']
    tab, t1, t2 = _tc_embed(h, x, p['emb_in_w'], p['emb_in_b'].reshape(1, 32),
                            e1w0[0:32], e1w0[32:64])
    hproj = None

    for l in range(3):
        gs = _sc_gather(t1, t2, row2, col2)
        e1w = p[f'e1_w_{l}']
        wr = e1w[64:65]
        r4 = jnp.concatenate([wr, wr, wr, zb], axis=0)
        mt = _tc_edge(
            gs, eap,
            r4, e1w[65:81],
            p[f'e1_b_{l}'].reshape(1, 32),
            p[f'e2_w_{l}'], p[f'e2_b_{l}'].reshape(1, 32),
            p[f'c1_w_{l}'], p[f'c1_b_{l}'].reshape(1, 32),
            p[f'c2_w_{l}'])
        accp = _sc_scatter(mt, row2, zrows)
        n1w = p[f'n1_w_{l}']
        if l < 2:
            wo, bo = eye, zb
            e1wn = p[f'e1_w_{l + 1}']
            wsn, wtn = e1wn[0:32], e1wn[32:64]
        else:
            wo, bo = p['emb_out_w'], p['emb_out_b'].reshape(1, 32)
            wsn, wtn = zw, zw
        tab, t1, t2, hproj = _tc_node(
            tab, accp[0], accp[1],
            n1w[0:32], n1w[32:64], p[f'n1_b_{l}'].reshape(1, 32),
            p[f'n2_w_{l}'], p[f'n2_b_{l}'].reshape(1, 32), wo, bo, wsn, wtn)

    return hproj, tab[:, 32:35]


# speedup vs baseline: 5.2254x; 1.0439x over previous
import functools

import jax
import jax.numpy as jnp
from jax import lax
from jax.experimental import pallas as pl
from jax.experimental.pallas import tpu as pltpu
from jax.experimental.pallas import tpu_sc as plsc

N = 50000
E = 800000
EP = 802816
NR = EP // 128
NC, NS = 2, 16
NW = NC * NS
RPW = NR // NW
G = 4
STEPS = RPW // G
GD = 48
MD = 40
NPS = N // NS
ZR = 125
BE = 4096
BN = 2000


def _mesh():
    return plsc.VectorSubcoreMesh(core_axis_name="c", subcore_axis_name="s")


_SC_PARAMS = functools.partial(
    pltpu.CompilerParams, use_tc_tiling_on_sc=False)


def _silu(v):
    return 0.5 * v * jnp.tanh(0.5 * v) + 0.5 * v


def _sc_gather(tab1, tab2, row2, col2):
    CE = G * 128

    def body(t1_r, t2_r, row_r, col_r, gs_o,
             rbuf0, cbuf0, rbuf1, cbuf1, grb0, gcb0, grb1, gcb1, sem0, sem1):
        wid = lax.axis_index("s") * NC + lax.axis_index("c")
        base = wid * RPW

        def fire(g, rbuf, cbuf, grb, gcb, sem):
            cid = base + g * G
            pltpu.sync_copy(row_r.at[pl.ds(cid, G)], rbuf)
            pltpu.sync_copy(col_r.at[pl.ds(cid, G)], cbuf)
            for j in range(G):
                pltpu.async_copy(
                    t1_r.at[rbuf.at[j]], grb.at[pl.ds(j * 128, 128)], sem)
                pltpu.async_copy(
                    t2_r.at[cbuf.at[j]], gcb.at[pl.ds(j * 128, 128)], sem)

        def drain(rbuf, cbuf, grb, gcb, sem):
            for j in range(G):
                pltpu.make_async_copy(
                    t1_r.at[rbuf.at[j]], grb.at[pl.ds(j * 128, 128)],
                    sem).wait()
                pltpu.make_async_copy(
                    t2_r.at[cbuf.at[j]], gcb.at[pl.ds(j * 128, 128)],
                    sem).wait()

        def addstore(g, grb, gcb):
            def add4(i, carry):
                for u in range(4):
                    for k in range(3):
                        sl = pl.ds(k * 16, 16)
                        grb[i * 4 + u, sl] = grb[i * 4 + u, sl] + \
                            gcb[i * 4 + u, sl]
                return carry

            lax.fori_loop(0, CE // 4, add4, 0)
            cid = base + g * G
            pltpu.sync_copy(grb, gs_o.at[pl.ds(cid * 128, CE)])

        fire(0, rbuf0, cbuf0, grb0, gcb0, sem0)

        def step(k, carry):
            g0 = k * 2
            drain(rbuf0, cbuf0, grb0, gcb0, sem0)
            fire(g0 + 1, rbuf1, cbuf1, grb1, gcb1, sem1)
            addstore(g0, grb0, gcb0)
            drain(rbuf1, cbuf1, grb1, gcb1, sem1)
            fire(g0 + 2, rbuf0, cbuf0, grb0, gcb0, sem0)
            addstore(g0 + 1, grb1, gcb1)
            return carry

        lax.fori_loop(0, (STEPS - 1) // 2, step, 0)
        drain(rbuf0, cbuf0, grb0, gcb0, sem0)
        addstore(STEPS - 1, grb0, gcb0)

    return pl.kernel(
        body,
        out_type=jax.ShapeDtypeStruct((EP, GD), jnp.float32),
        mesh=_mesh(),
        compiler_params=_SC_PARAMS(),
        scratch_types=[
            pltpu.VMEM((G, 128), jnp.int32),
            pltpu.VMEM((G, 128), jnp.int32),
            pltpu.VMEM((G, 128), jnp.int32),
            pltpu.VMEM((G, 128), jnp.int32),
            pltpu.VMEM((CE, GD), jnp.float32),
            pltpu.VMEM((CE, GD), jnp.float32),
            pltpu.VMEM((CE, GD), jnp.float32),
            pltpu.VMEM((CE, GD), jnp.float32),
            pltpu.SemaphoreType.DMA,
            pltpu.SemaphoreType.DMA,
        ],
    )(tab1, tab2, row2, col2)


def _sc_scatter(mt, row2, zrows):
    def body(mt_r, row_r, z_r, out_r, acc, rb0, rb1, mtb, sem0, sem1):
        c = lax.axis_index("c")
        s = lax.axis_index("s")
        wid = s * NC + c
        r0 = s * NPS

        pltpu.sync_copy(z_r, mtb.at[pl.ds(0, ZR)])

        def zstep(k, carry):
            pltpu.sync_copy(mtb.at[pl.ds(0, ZR)],
                            acc.at[pl.ds(r0 + k * ZR, ZR)])
            return carry

        lax.fori_loop(0, NPS // ZR, zstep, 0)
        plsc.subcore_barrier()

        base = wid * RPW
        last = base + RPW - 1

        def ifire(cid, rb, sem):
            pltpu.async_copy(row_r.at[pl.ds(cid, 1)], rb, sem)

        def iwait(rb, sem):
            pltpu.make_async_copy(row_r.at[pl.ds(0, 1)], rb, sem).wait()

        def work(cid, rb):
            pltpu.sync_copy(mt_r.at[pl.ds(cid * 128, 128)], mtb)
            pltpu.sync_copy(mtb, acc.at[rb.at[0]], add=True)

        ifire(base, rb0, sem0)

        def step(k, carry):
            g0 = base + k * 2
            ifire(g0 + 1, rb1, sem1)
            iwait(rb0, sem0)
            work(g0, rb0)
            ifire(jnp.minimum(g0 + 2, last), rb0, sem0)
            iwait(rb1, sem1)
            work(g0 + 1, rb1)
            return carry

        lax.fori_loop(0, RPW // 2, step, 0)
        iwait(rb0, sem0)
        plsc.subcore_barrier()

        def ostep(k, carry):
            pltpu.sync_copy(acc.at[pl.ds(r0 + k * ZR, ZR)],
                            mtb.at[pl.ds(0, ZR)])
            pltpu.sync_copy(mtb.at[pl.ds(0, ZR)],
                            out_r.at[c, pl.ds(r0 + k * ZR, ZR)])
            return carry

        lax.fori_loop(0, NPS // ZR, ostep, 0)

    return pl.kernel(
        body,
        out_type=jax.ShapeDtypeStruct((NC, N, MD), jnp.float32),
        mesh=_mesh(),
        compiler_params=_SC_PARAMS(),
        scratch_types=[
            pltpu.VMEM_SHARED((N, MD), jnp.float32),
            pltpu.VMEM((1, 128), jnp.int32),
            pltpu.VMEM((1, 128), jnp.int32),
            pltpu.VMEM((128, MD), jnp.float32),
            pltpu.SemaphoreType.DMA,
            pltpu.SemaphoreType.DMA,
        ],
    )(mt, row2, zrows)


def _tc_embed(h, x, w, b, ws, wt):
    def body(h_r, x_r, w_r, b_r, ws_r, wt_r, tab_o, t1_o, t2_o):
        h0 = jnp.dot(h_r[:], w_r[:],
                     preferred_element_type=jnp.float32) + b_r[:]
        xv = x_r[:]
        z1 = jnp.zeros((BN, 1), jnp.float32)
        z13 = jnp.zeros((BN, 13), jnp.float32)
        tab_o[:] = jnp.concatenate([h0, xv, z1], axis=1)
        t1_o[:] = jnp.concatenate(
            [jnp.dot(h0, ws_r[:], preferred_element_type=jnp.float32),
             xv, z13], axis=1)
        t2_o[:] = jnp.concatenate(
            [jnp.dot(h0, wt_r[:], preferred_element_type=jnp.float32),
             -xv, z13], axis=1)

    return pl.pallas_call(
        body,
        grid=(N // BN,),
        in_specs=[
            pl.BlockSpec((BN, 125), lambda i: (i, 0)),
            pl.BlockSpec((BN, 3), lambda i: (i, 0)),
            pl.BlockSpec((125, 32), lambda i: (0, 0)),
            pl.BlockSpec((1, 32), lambda i: (0, 0)),
            pl.BlockSpec((32, 32), lambda i: (0, 0)),
            pl.BlockSpec((32, 32), lambda i: (0, 0)),
        ],
        out_specs=(pl.BlockSpec((BN, 36), lambda i: (i, 0)),
                   pl.BlockSpec((BN, GD), lambda i: (i, 0)),
                   pl.BlockSpec((BN, GD), lambda i: (i, 0))),
        out_shape=(jax.ShapeDtypeStruct((N, 36), jnp.float32),
                   jax.ShapeDtypeStruct((N, GD), jnp.float32),
                   jax.ShapeDtypeStruct((N, GD), jnp.float32)),
    )(h, x, w, b, ws, wt)


def _tc_edge(gs, ea, r4, we, b1, e2, b2, c1, c1b, c2):
    def body(gs_r, ea_r, r4_r, we_r, b1_r, e2_r,
             b2_r, c1_r, c1b_r, c2_r, mt_o):
        pid = pl.program_id(0)
        g = gs_r[:]
        d = g[:, 32:36]
        dsq = d * d
        pre = (g[:, 0:32]
               + jnp.dot(ea_r[:], we_r[:], preferred_element_type=jnp.float32)
               + jnp.dot(dsq, r4_r[:], preferred_element_type=jnp.float32)
               + b1_r[:])
        m1 = _silu(pre)
        m = _silu(jnp.dot(m1, e2_r[:],
                          preferred_element_type=jnp.float32) + b2_r[:])
        t = _silu(jnp.dot(m, c1_r[:],
                          preferred_element_type=jnp.float32) + c1b_r[:])
        sc = jnp.dot(t, c2_r[:], preferred_element_type=jnp.float32)
        ids = pid * BE + lax.broadcasted_iota(jnp.int32, (BE, 1), 0)
        validf = (ids < E).astype(jnp.float32)
        scv = sc * validf
        mt_o[:, 0:32] = m * validf
        mt_o[:, 32:36] = d * scv
        mt_o[:, 36:37] = validf
        mt_o[:, 37:40] = jnp.zeros((BE, 3), jnp.float32)

    full = lambda shape: pl.BlockSpec(shape, lambda i: (0, 0))
    return pl.pallas_call(
        body,
        grid=(EP // BE,),
        in_specs=[
            pl.BlockSpec((BE, GD), lambda i: (i, 0)),
            pl.BlockSpec((BE, 16), lambda i: (i, 0)),
            full((4, 32)), full((16, 32)), full((1, 32)), full((32, 32)),
            full((1, 32)), full((32, 32)), full((1, 32)), full((32, 1)),
        ],
        out_specs=pl.BlockSpec((BE, MD), lambda i: (i, 0)),
        out_shape=jax.ShapeDtypeStruct((EP, MD), jnp.float32),
    )(gs, ea, r4, we, b1, e2, b2, c1, c1b, c2)


def _tc_node(tab, a0, a1, n1h, n1m, n1b, n2, n2b, wo, bo, wsn, wtn):
    def body(tab_r, a0_r, a1_r, n1h_r, n1m_r, n1b_r, n2_r, n2b_r,
             wo_r, bo_r, wsn_r, wtn_r, tab_o, t1_o, t2_o, hp_o):
        h = tab_r[:, 0:32]
        x4 = tab_r[:, 32:36]
        magg = a0_r[:, 0:32] + a1_r[:, 0:32]
        tr = a0_r[:, 32:36] + a1_r[:, 32:36]
        cnt = a0_r[:, 36:37] + a1_r[:, 36:37]
        o = _silu(jnp.dot(h, n1h_r[:], preferred_element_type=jnp.float32)
                  + jnp.dot(magg, n1m_r[:], preferred_element_type=jnp.float32)
                  + n1b_r[:])
        hn = h + jnp.dot(o, n2_r[:],
                         preferred_element_type=jnp.float32) + n2b_r[:]
        xn = x4 + tr / jnp.clip(cnt, 1.0, None)
        xn3 = xn[:, 0:3]
        z13 = jnp.zeros((BN, 13), jnp.float32)
        tab_o[:] = jnp.concatenate([hn, xn], axis=1)
        t1_o[:] = jnp.concatenate(
            [jnp.dot(hn, wsn_r[:], preferred_element_type=jnp.float32),
             xn3, z13], axis=1)
        t2_o[:] = jnp.concatenate(
            [jnp.dot(hn, wtn_r[:], preferred_element_type=jnp.float32),
             -xn3, z13], axis=1)
        hp_o[:] = jnp.dot(hn, wo_r[:],
                          preferred_element_type=jnp.float32) + bo_r[:]

    full = lambda shape: pl.BlockSpec(shape, lambda i: (0, 0))
    return pl.pallas_call(
        body,
        grid=(N // BN,),
        in_specs=[
            pl.BlockSpec((BN, 36), lambda i: (i, 0)),
            pl.BlockSpec((BN, MD), lambda i: (i, 0)),
            pl.BlockSpec((BN, MD), lambda i: (i, 0)),
            full((32, 32)), full((32, 32)), full((1, 32)),
            full((32, 32)), full((1, 32)), full((32, 32)), full((1, 32)),
            full((32, 32)), full((32, 32)),
        ],
        out_specs=(pl.BlockSpec((BN, 36), lambda i: (i, 0)),
                   pl.BlockSpec((BN, GD), lambda i: (i, 0)),
                   pl.BlockSpec((BN, GD), lambda i: (i, 0)),
                   pl.BlockSpec((BN, 32), lambda i: (i, 0))),
        out_shape=(jax.ShapeDtypeStruct((N, 36), jnp.float32),
                   jax.ShapeDtypeStruct((N, GD), jnp.float32),
                   jax.ShapeDtypeStruct((N, GD), jnp.float32),
                   jax.ShapeDtypeStruct((N, 32), jnp.float32)),
    )(tab, a0, a1, n1h, n1m, n1b, n2, n2b, wo, bo, wsn, wtn)


def kernel(h, x, edge_index, edge_attr, params):
    p = params
    row = edge_index[0]
    col = edge_index[1]
    row2 = jnp.pad(row, (0, EP - E)).reshape(NR, 128)
    col2 = jnp.pad(col, (0, EP - E)).reshape(NR, 128)
    eap = jnp.pad(edge_attr, ((0, EP - E), (0, 0)))
    zrows = jnp.zeros((ZR, MD), jnp.float32)
    eye = jnp.eye(32, dtype=jnp.float32)
    zb = jnp.zeros((1, 32), jnp.float32)
    zw = jnp.zeros((32, 32), jnp.float32)

    e1w0 = p['e1_w_0']
    tab, t1, t2 = _tc_embed(h, x, p['emb_in_w'], p['emb_in_b'].reshape(1, 32),
                            e1w0[0:32], e1w0[32:64])
    hproj = None

    for l in range(3):
        gs = _sc_gather(t1, t2, row2, col2)
        e1w = p[f'e1_w_{l}']
        wr = e1w[64:65]
        r4 = jnp.concatenate([wr, wr, wr, zb], axis=0)
        mt = _tc_edge(
            gs, eap,
            r4, e1w[65:81],
            p[f'e1_b_{l}'].reshape(1, 32),
            p[f'e2_w_{l}'], p[f'e2_b_{l}'].reshape(1, 32),
            p[f'c1_w_{l}'], p[f'c1_b_{l}'].reshape(1, 32),
            p[f'c2_w_{l}'])
        accp = _sc_scatter(mt, row2, zrows)
        n1w = p[f'n1_w_{l}']
        if l < 2:
            wo, bo = eye, zb
            e1wn = p[f'e1_w_{l + 1}']
            wsn, wtn = e1wn[0:32], e1wn[32:64]
        else:
            wo, bo = p['emb_out_w'], p['emb_out_b'].reshape(1, 32)
            wsn, wtn = zw, zw
        tab, t1, t2, hproj = _tc_node(
            tab, accp[0], accp[1],
            n1w[0:32], n1w[32:64], p[f'n1_b_{l}'].reshape(1, 32),
            p[f'n2_w_{l}'], p[f'n2_b_{l}'].reshape(1, 32), wo, bo, wsn, wtn)

    return hproj, tab[:, 32:35]
